# baseline (device time: 354702 ns/iter reference)
import jax
import jax.numpy as jnp
from jax import lax
from jax.experimental import pallas as pl
from jax.experimental.pallas import tpu as pltpu

jax.config.update("jax_compilation_cache_dir", "/tmp/scband_jax_cache")
jax.config.update("jax_persistent_cache_min_compile_time_secs", 5.0)

N_DEV = 4
M_PER = 2048
K_PER = 2048
K = 8192
N = 4096
MH = M_PER // 2
CH = K_PER // 2
NT = 512
N_NT = N // NT


def _gelu(y):
    c = 0.7978845608028654
    return 0.5 * y * (1.0 + jnp.tanh(c * (y + 0.044715 * y**3)))


def _fused_body(
    x_ref,
    xb_ref,
    w_ref,
    out_ref,
    transit_ref,
    recv_ref,
    acc_ref,
    lhs_ref,
    wbuf_ref,
    send_sems,
    recv_sems,
    rt_send_sems,
    rt_recv_sems,
    f_send_sems,
    d_recv_sems,
    x_sem,
    w_sems,
    out_sem,
):
    p = lax.axis_index("i")
    left = lax.rem(p + N_DEV - 1, N_DEV)
    right = lax.rem(p + 1, N_DEV)
    diag = lax.rem(p + 2, N_DEV)

    barrier_sem = pltpu.get_barrier_semaphore()
    for nbr in (left, right):
        pl.semaphore_signal(
            barrier_sem,
            inc=1,
            device_id=(nbr,),
            device_id_type=pl.DeviceIdType.MESH,
        )
    pl.semaphore_wait(barrier_sem, 2)

    rdmas = []

    def _send(src, dst, ssem, rsem, dev):
        rdma = pltpu.make_async_remote_copy(
            src_ref=src,
            dst_ref=dst,
            send_sem=ssem,
            recv_sem=rsem,
            device_id=(dev,),
            device_id_type=pl.DeviceIdType.MESH,
        )
        rdma.start()
        rdmas.append(rdma)

    for mh in range(2):
        for d, q in ((1, right), (3, left)):
            _send(
                xb_ref.at[pl.ds(q * M_PER + mh * MH, MH), :],
                recv_ref.at[d - 1, pl.ds(mh * MH, MH), :],
                send_sems.at[d - 1, mh],
                recv_sems.at[d - 1, mh],
                q,
            )
        for side, q, c0 in ((0, right, CH), (1, left, 0)):
            _send(
                xb_ref.at[pl.ds(diag * M_PER + mh * MH, MH), pl.ds(c0, CH)],
                transit_ref.at[side, mh],
                rt_send_sems.at[side, mh],
                rt_recv_sems.at[side, mh],
                q,
            )

    def _forward(mh):
        for side, q, chalf in ((0, right, 1), (1, left, 0)):
            pltpu.make_async_remote_copy(
                src_ref=xb_ref.at[pl.ds(0, MH), pl.ds(0, CH)],
                dst_ref=transit_ref.at[side, mh],
                send_sem=rt_send_sems.at[side, mh],
                recv_sem=rt_recv_sems.at[side, mh],
                device_id=(p,),
                device_id_type=pl.DeviceIdType.MESH,
            ).wait_recv()
            _send(
                transit_ref.at[side, mh],
                recv_ref.at[1, pl.ds(mh * MH, MH), pl.ds(chalf * CH, CH)],
                f_send_sems.at[side, mh],
                d_recv_sems.at[chalf, mh],
                q,
            )

    slot_of = [None, 0, 2, 1]
    ks = [p, left, right, diag]

    w_copies = []
    for mh in range(2):
        for s in range(4):
            for n in range(N_NT):
                i = len(w_copies)
                w_copies.append(
                    pltpu.make_async_copy(
                        w_ref.at[pl.ds(ks[s] * K_PER, K_PER), pl.ds(n * NT, NT)],
                        wbuf_ref.at[i % 2],
                        w_sems.at[i % 2],
                    )
                )
    w_copies[0].start()
    w_copies[1].start()

    out_cps = []
    i = 0
    for mh in range(2):
        if mh == 1:
            out_cps[0].wait()
        for s in range(4):
            if s == 0:
                local_cp = pltpu.make_async_copy(
                    x_ref.at[pl.ds(p * M_PER + mh * MH, MH), :],
                    lhs_ref,
                    x_sem,
                )
                local_cp.start()
                local_cp.wait()
            elif s < 3:
                slot = slot_of[s]
                pltpu.make_async_remote_copy(
                    src_ref=xb_ref.at[pl.ds(0, MH), :],
                    dst_ref=recv_ref.at[slot, pl.ds(mh * MH, MH), :],
                    send_sem=send_sems.at[slot, mh],
                    recv_sem=recv_sems.at[slot, mh],
                    device_id=(p,),
                    device_id_type=pl.DeviceIdType.MESH,
                ).wait_recv()
                lhs_ref[...] = recv_ref[
                    slot, mh * MH : (mh + 1) * MH, :
                ].astype(jnp.float32)
            else:
                for chalf in range(2):
                    pltpu.make_async_remote_copy(
                        src_ref=xb_ref.at[pl.ds(0, MH), pl.ds(0, CH)],
                        dst_ref=recv_ref.at[
                            1, pl.ds(mh * MH, MH), pl.ds(chalf * CH, CH)
                        ],
                        send_sem=f_send_sems.at[chalf, mh],
                        recv_sem=d_recv_sems.at[chalf, mh],
                        device_id=(p,),
                        device_id_type=pl.DeviceIdType.MESH,
                    ).wait_recv()
                lhs_ref[...] = recv_ref[
                    1, mh * MH : (mh + 1) * MH, :
                ].astype(jnp.float32)
            for n in range(N_NT):
                w_copies[i].wait()
                partial = jnp.dot(
                    lhs_ref[...],
                    wbuf_ref[i % 2],
                    preferred_element_type=jnp.float32,
                )
                nsl = slice(n * NT, (n + 1) * NT)
                if s == 0:
                    acc_ref[:, nsl] = partial
                else:
                    acc_ref[:, nsl] += partial
                if i + 2 < len(w_copies):
                    w_copies[i + 2].start()
                i += 1
            if mh == 0 and s in (0, 2):
                _forward(0 if s == 0 else 1)
        for n in range(N_NT):
            nsl = slice(n * NT, (n + 1) * NT)
            acc_ref[:, nsl] = _gelu(acc_ref[:, nsl])
        out_cp = pltpu.make_async_copy(
            acc_ref, out_ref.at[pl.ds(mh * MH, MH), :], out_sem
        )
        out_cp.start()
        out_cps.append(out_cp)

    out_cps[1].wait()
    for rdma in rdmas:
        rdma.wait_send()


def _fused(x, xb, w):
    out, _ = pl.pallas_call(
        _fused_body,
        out_shape=[
            jax.ShapeDtypeStruct((M_PER, N), jnp.float32),
            jax.ShapeDtypeStruct((2, 2, MH, CH), jnp.bfloat16),
        ],
        in_specs=[
            pl.BlockSpec(memory_space=pl.ANY),
            pl.BlockSpec(memory_space=pl.ANY),
            pl.BlockSpec(memory_space=pl.ANY),
        ],
        out_specs=[
            pl.BlockSpec(memory_space=pl.ANY),
            pl.BlockSpec(memory_space=pl.ANY),
        ],
        scratch_shapes=[
            pltpu.VMEM((N_DEV - 1, M_PER, K_PER), jnp.bfloat16),
            pltpu.VMEM((MH, N), jnp.float32),
            pltpu.VMEM((MH, K_PER), jnp.float32),
            pltpu.VMEM((2, K_PER, NT), jnp.float32),
            pltpu.SemaphoreType.DMA((N_DEV - 1, 2)),
            pltpu.SemaphoreType.DMA((N_DEV - 1, 2)),
            pltpu.SemaphoreType.DMA((2, 2)),
            pltpu.SemaphoreType.DMA((2, 2)),
            pltpu.SemaphoreType.DMA((2, 2)),
            pltpu.SemaphoreType.DMA((2, 2)),
            pltpu.SemaphoreType.DMA,
            pltpu.SemaphoreType.DMA((2,)),
            pltpu.SemaphoreType.DMA,
        ],
        compiler_params=pltpu.CompilerParams(
            collective_id=0,
            vmem_limit_bytes=65472 * 1024,
        ),
    )(x, xb, w)
    return out


def kernel(x, w_mat):
    return _fused(x, x.astype(jnp.bfloat16), w_mat)


# device time: 310498 ns/iter; 1.1424x vs baseline; 1.1424x over previous
import jax
import jax.numpy as jnp
from jax import lax
from jax.experimental import pallas as pl
from jax.experimental.pallas import tpu as pltpu

jax.config.update("jax_compilation_cache_dir", "/tmp/scband_jax_cache")
jax.config.update("jax_persistent_cache_min_compile_time_secs", 5.0)

N_DEV = 4
M_PER = 2048
K_PER = 2048
K = 8192
N = 4096
MH = M_PER // 2
NT = 512
N_NT = N // NT


def _gelu(y):
    c = 0.7978845608028654
    return 0.5 * y * (1.0 + jnp.tanh(c * (y + 0.044715 * y**3)))


def _fused_body(
    x_ref,
    xb_ref,
    w_ref,
    out_ref,
    recv_ref,
    acc_ref,
    lhs_ref,
    wbuf_ref,
    send_sems,
    recv_sems,
    x_sem,
    w_sems,
    out_sem,
):
    p = lax.axis_index("i")

    barrier_sem = pltpu.get_barrier_semaphore()
    for d in range(1, N_DEV):
        peer = lax.rem(p + d, N_DEV)
        pl.semaphore_signal(
            barrier_sem,
            inc=1,
            device_id=(peer,),
            device_id_type=pl.DeviceIdType.MESH,
        )
    pl.semaphore_wait(barrier_sem, N_DEV - 1)

    rdmas = []
    for mh in range(2):
        for d in range(1, N_DEV):
            q = lax.rem(p + d, N_DEV)
            rdma = pltpu.make_async_remote_copy(
                src_ref=xb_ref.at[pl.ds(q * M_PER + mh * MH, MH), :],
                dst_ref=recv_ref.at[d - 1, pl.ds(mh * MH, MH), :],
                send_sem=send_sems.at[d - 1, mh],
                recv_sem=recv_sems.at[d - 1, mh],
                device_id=(q,),
                device_id_type=pl.DeviceIdType.MESH,
            )
            rdma.start()
            rdmas.append(rdma)

    slot_of = [None, 0, 2, 1]
    ks = [p] + [
        lax.rem(p + (N_DEV - d), N_DEV) for d in (1, 3, 2)
    ]

    w_copies = []
    for mh in range(2):
        for s in range(4):
            for n in range(N_NT):
                i = len(w_copies)
                w_copies.append(
                    pltpu.make_async_copy(
                        w_ref.at[pl.ds(ks[s] * K_PER, K_PER), pl.ds(n * NT, NT)],
                        wbuf_ref.at[i % 2],
                        w_sems.at[i % 2],
                    )
                )
    w_copies[0].start()
    w_copies[1].start()

    out_cps = []
    i = 0
    for mh in range(2):
        if mh == 1:
            out_cps[0].wait()
        for s in range(4):
            if s == 0:
                local_cp = pltpu.make_async_copy(
                    x_ref.at[pl.ds(p * M_PER + mh * MH, MH), :],
                    lhs_ref,
                    x_sem,
                )
                local_cp.start()
                local_cp.wait()
            else:
                slot = slot_of[s]
                pltpu.make_async_remote_copy(
                    src_ref=xb_ref.at[pl.ds(0, MH), :],
                    dst_ref=recv_ref.at[slot, pl.ds(mh * MH, MH), :],
                    send_sem=send_sems.at[slot, mh],
                    recv_sem=recv_sems.at[slot, mh],
                    device_id=(p,),
                    device_id_type=pl.DeviceIdType.MESH,
                ).wait_recv()
                lhs_ref[...] = recv_ref[
                    slot, mh * MH : (mh + 1) * MH, :
                ].astype(jnp.float32)
            for n in range(N_NT):
                w_copies[i].wait()
                partial = jnp.dot(
                    lhs_ref[...],
                    wbuf_ref[i % 2],
                    preferred_element_type=jnp.float32,
                )
                nsl = slice(n * NT, (n + 1) * NT)
                if s == 0:
                    acc_ref[:, nsl] = partial
                else:
                    acc_ref[:, nsl] += partial
                if i + 2 < len(w_copies):
                    w_copies[i + 2].start()
                i += 1
        for n in range(N_NT):
            nsl = slice(n * NT, (n + 1) * NT)
            acc_ref[:, nsl] = _gelu(acc_ref[:, nsl])
        out_cp = pltpu.make_async_copy(
            acc_ref, out_ref.at[pl.ds(mh * MH, MH), :], out_sem
        )
        out_cp.start()
        out_cps.append(out_cp)

    out_cps[1].wait()
    for rdma in rdmas:
        rdma.wait_send()


def _fused(x, xb, w):
    return pl.pallas_call(
        _fused_body,
        out_shape=jax.ShapeDtypeStruct((M_PER, N), jnp.float32),
        in_specs=[
            pl.BlockSpec(memory_space=pl.ANY),
            pl.BlockSpec(memory_space=pl.ANY),
            pl.BlockSpec(memory_space=pl.ANY),
        ],
        out_specs=pl.BlockSpec(memory_space=pl.ANY),
        scratch_shapes=[
            pltpu.VMEM((N_DEV - 1, M_PER, K_PER), jnp.bfloat16),
            pltpu.VMEM((MH, N), jnp.float32),
            pltpu.VMEM((MH, K_PER), jnp.float32),
            pltpu.VMEM((2, K_PER, NT), jnp.float32),
            pltpu.SemaphoreType.DMA((N_DEV - 1, 2)),
            pltpu.SemaphoreType.DMA((N_DEV - 1, 2)),
            pltpu.SemaphoreType.DMA,
            pltpu.SemaphoreType.DMA((2,)),
            pltpu.SemaphoreType.DMA,
        ],
        compiler_params=pltpu.CompilerParams(
            collective_id=0,
            vmem_limit_bytes=65472 * 1024,
        ),
    )(x, xb, w)


def kernel(x, w_mat):
    return _fused(x, x.astype(jnp.bfloat16), w_mat)
